# Initial kernel scaffold; baseline (speedup 1.0000x reference)
import jax
import jax.numpy as jnp
from jax.experimental import pallas as pl


def kernel(image, gather_indices):
    raise NotImplementedError("write your pallas kernel here")


# speedup vs baseline: 1.5838x; 1.5838x over previous
import functools

import jax
import jax.numpy as jnp
from jax import lax
from jax.experimental import pallas as pl
from jax.experimental.pallas import tpu as pltpu
from jax.experimental.pallas import tpu_sc as plsc

NW = 32
CHUNK = 128
NBUF = 8


@functools.lru_cache(maxsize=None)
def _build(B, D):
    assert B % (NW * CHUNK) == 0
    nchunk = B // (NW * CHUNK)
    assert nchunk % NBUF == 0
    ngroups = nchunk // NBUF

    mesh = plsc.VectorSubcoreMesh(core_axis_name="c", subcore_axis_name="s")

    @functools.partial(
        pl.kernel,
        out_type=jax.ShapeDtypeStruct((B, D), jnp.float32),
        mesh=mesh,
        scratch_types=[
            pltpu.VMEM((nchunk, CHUNK), jnp.int32),
            pltpu.VMEM((NBUF, CHUNK, D), jnp.float32),
            pltpu.SemaphoreType.DMA((NBUF,)),
            pltpu.SemaphoreType.DMA((NBUF,)),
        ],
        compiler_params=pltpu.CompilerParams(use_tc_tiling_on_sc=False),
    )
    def gather_kernel(table, idx_hbm, out_hbm, idx_v, rows, gsem, ssem):
        w = lax.axis_index("s") * 2 + lax.axis_index("c")
        pltpu.sync_copy(idx_hbm.at[pl.ds(w * nchunk, nchunk), :], idx_v)
        out_base = w * (nchunk * CHUNK)

        def start_gather(j, b):
            pltpu.async_copy(table.at[idx_v.at[j]], rows.at[b], gsem.at[b])

        def wait_gather(b):
            pltpu.make_async_copy(table.at[idx_v.at[0]], rows.at[b],
                                  gsem.at[b]).wait()

        def out_slice(j):
            return out_hbm.at[pl.ds(out_base + j * CHUNK, CHUNK), :]

        def start_scatter(j, b):
            pltpu.async_copy(rows.at[b], out_slice(j), ssem.at[b])

        def wait_scatter(j, b):
            pltpu.make_async_copy(rows.at[b], out_slice(j), ssem.at[b]).wait()

        for b in range(NBUF):
            start_gather(b, b)

        @pl.loop(0, ngroups - 1)
        def _(g):
            for b in range(NBUF):
                j = g * NBUF + b
                wait_gather(b)
                start_scatter(j, b)
                wait_scatter(j, b)
                start_gather(j + NBUF, b)

        for b in range(NBUF):
            j = (ngroups - 1) * NBUF + b
            wait_gather(b)
            start_scatter(j, b)
            wait_scatter(j, b)

    return gather_kernel


def kernel(image, gather_indices):
    nb, nf, _ = gather_indices.shape
    B = nb * nf
    D = image.shape[1]
    idx = gather_indices.reshape(B // CHUNK, CHUNK).astype(jnp.int32)
    out = _build(B, D)(image, idx)
    return out.reshape(nb, nf, D)
